# baseline (speedup 1.0000x reference)
import functools

import jax
import jax.numpy as jnp
from jax import lax
from jax.experimental import pallas as pl
from jax.experimental.pallas import tpu as pltpu
from jax.experimental.pallas import tpu_sc as plsc

_NUM = 13
_CAT = 26
_V = 100000
_VP = 100096
_D = 32
_F = _NUM + _CAT

_NC = 2
_NS = 16
_NW = _NC * _NS
_NB = 128
_VCH = 50176
_NVB = (_V + _VCH - 1) // _VCH
_VP2 = _NVB * _VCH


def _tc_transpose_body(x_ref, o_ref):
  x = x_ref[0]
  eye = (lax.broadcasted_iota(jnp.int32, (_D, _D), 0) ==
         lax.broadcasted_iota(jnp.int32, (_D, _D), 1)).astype(jnp.float32)
  o_ref[0] = lax.dot_general(x, eye, (((0,), (0,)), ((), ())),
                             preferred_element_type=jnp.float32)


def _tc_relayout(tab_t):
  ncat = tab_t.shape[0]
  return pl.pallas_call(
      _tc_transpose_body,
      grid=(ncat, _NVB),
      in_specs=[pl.BlockSpec((1, _D, _VCH), lambda c, j: (c, 0, j))],
      out_specs=pl.BlockSpec((1, _VCH, _D), lambda c, j: (c, j, 0)),
      out_shape=jax.ShapeDtypeStruct((tab_t.shape[0], _VP2, _D),
                                     jnp.float32),
      compiler_params=pltpu.CompilerParams(vmem_limit_bytes=100 * 1024 * 1024),
  )(tab_t)


_SPLIT = 13


def _gather_body(x_hbm, tabx_hbm, taby_hbm, out_hbm, xb, idx_v, gbuf, outb, sem):
  wid = lax.axis_index("s") * _NC + lax.axis_index("c")
  b = x_hbm.shape[0]
  b_per_w = b // _NW
  nblk = b_per_w // _NB

  def blk_body(blk, _):
    base = wid * b_per_w + blk * _NB
    pltpu.sync_copy(x_hbm.at[pl.ds(base, _NB), :], xb)
    lanes = lax.iota(jnp.int32, 16)
    for c in range(_CAT):
      col = jnp.full((16,), _NUM + c, jnp.int32)
      for j in range(_NB // 16):
        v = plsc.load_gather(xb, [lanes + (j * 16), col])
        idx_v[c, pl.ds(j * 16, 16)] = v.astype(jnp.int32)
    descs = []
    for c in range(_CAT):
      tab = tabx_hbm.at[c] if c < _SPLIT else taby_hbm.at[c - _SPLIT]
      descs.append(
          pltpu.async_copy(tab.at[idx_v.at[c]], gbuf.at[c], sem))
    for d in descs:
      d.wait()
    def row_body(r, _):
      a0 = gbuf[0, r, pl.ds(0, 16)]
      a1 = gbuf[0, r, pl.ds(16, 16)]
      for c in range(1, _CAT):
        a0 = a0 + gbuf[c, r, pl.ds(0, 16)]
        a1 = a1 + gbuf[c, r, pl.ds(16, 16)]
      outb[r, pl.ds(0, 16)] = a0
      outb[r, pl.ds(16, 16)] = a1
      return 0
    lax.fori_loop(0, _NB, row_body, 0)
    pltpu.sync_copy(outb, out_hbm.at[pl.ds(base, _NB)])
    return 0

  lax.fori_loop(0, nblk, blk_body, 0)


def _sc_gather_sum(x, tabx, taby):
  b = x.shape[0]
  mesh = plsc.VectorSubcoreMesh(core_axis_name="c", subcore_axis_name="s")
  return pl.kernel(
      _gather_body,
      out_type=jax.ShapeDtypeStruct((b, _D), jnp.float32),
      mesh=mesh,
      scratch_types=[
          pltpu.VMEM((_NB, _F), jnp.float32),
          pltpu.VMEM((_CAT, _NB), jnp.int32),
          pltpu.VMEM((_CAT, _NB, _D), jnp.float32),
          pltpu.VMEM((_NB, _D), jnp.float32),
          pltpu.SemaphoreType.DMA,
      ],
      compiler_params=pltpu.CompilerParams(
          use_tc_tiling_on_sc=False, needs_layout_passes=False),
  )(x, tabx, taby)


def _tc_body(xn_ref, w_ref, b_ref, cs_ref, o_ref):
  xn = xn_ref[...]
  w = w_ref[...]
  bias_sum = jnp.sum(b_ref[...], axis=0, keepdims=True)
  num = jnp.dot(xn, w, preferred_element_type=jnp.float32)
  o_ref[...] = (num + bias_sum + cs_ref[...]) * (1.0 / _F)


def _tc_finalize(x_num, num_weights, num_biases, cat_sum):
  b = x_num.shape[0]
  bt = 4096
  grid = b // bt
  return pl.pallas_call(
      _tc_body,
      grid=(grid,),
      in_specs=[
          pl.BlockSpec((bt, _NUM), lambda i: (i, 0)),
          pl.BlockSpec((_NUM, _D), lambda i: (0, 0)),
          pl.BlockSpec((_NUM, _D), lambda i: (0, 0)),
          pl.BlockSpec((bt, _D), lambda i: (i, 0)),
      ],
      out_specs=pl.BlockSpec((bt, _D), lambda i: (i, 0)),
      out_shape=jax.ShapeDtypeStruct((b, _D), jnp.float32),
  )(x_num, num_weights, num_biases, cat_sum)


@jax.jit
def kernel(x, tables, num_weights, num_biases):
  x_num = x[:, :_NUM]
  tabx = tables[:13]
  taby = _tc_relayout(jnp.transpose(tables[13:], (0, 2, 1)))
  cat_sum = _sc_gather_sum(x, tabx, taby)
  return _tc_finalize(x_num, num_weights, num_biases, cat_sum)


# speedup vs baseline: 1.4478x; 1.4478x over previous
import functools

import jax
import jax.numpy as jnp
from jax import lax
from jax.experimental import pallas as pl
from jax.experimental.pallas import tpu as pltpu
from jax.experimental.pallas import tpu_sc as plsc

_NUM = 13
_CAT = 26
_V = 100000
_D = 32
_F = _NUM + _CAT

_NC = 2
_NS = 16
_NW = _NC * _NS
_NB = 128


def _gather_body(x_hbm, tab_hbm, out_hbm, xb, idx_v, gbuf, outb, sem):
  wid = lax.axis_index("s") * _NC + lax.axis_index("c")
  b = x_hbm.shape[0]
  b_per_w = b // _NW
  nblk = b_per_w // _NB

  def blk_body(blk, _):
    base = wid * b_per_w + blk * _NB
    pltpu.sync_copy(x_hbm.at[pl.ds(base, _NB), :], xb)
    lanes = lax.iota(jnp.int32, 16)
    for c in range(_CAT):
      col = jnp.full((16,), _NUM + c, jnp.int32)
      for j in range(_NB // 16):
        v = plsc.load_gather(xb, [lanes + (j * 16), col])
        idx_v[c, pl.ds(j * 16, 16)] = v.astype(jnp.int32)
    descs = []
    for c in range(_CAT):
      descs.append(
          pltpu.async_copy(tab_hbm.at[c].at[idx_v.at[c]], gbuf.at[c], sem))
    for d in descs:
      d.wait()
    def row_body(r, _):
      a0 = gbuf[0, r, pl.ds(0, 16)]
      a1 = gbuf[0, r, pl.ds(16, 16)]
      for c in range(1, _CAT):
        a0 = a0 + gbuf[c, r, pl.ds(0, 16)]
        a1 = a1 + gbuf[c, r, pl.ds(16, 16)]
      outb[r, pl.ds(0, 16)] = a0
      outb[r, pl.ds(16, 16)] = a1
      return 0
    lax.fori_loop(0, _NB, row_body, 0)
    pltpu.sync_copy(outb, out_hbm.at[pl.ds(base, _NB)])
    return 0

  lax.fori_loop(0, nblk, blk_body, 0)


def _sc_gather_sum(x, tables):
  b = x.shape[0]
  mesh = plsc.VectorSubcoreMesh(core_axis_name="c", subcore_axis_name="s")
  return pl.kernel(
      _gather_body,
      out_type=jax.ShapeDtypeStruct((b, _D), jnp.float32),
      mesh=mesh,
      scratch_types=[
          pltpu.VMEM((_NB, _F), jnp.float32),
          pltpu.VMEM((_CAT, _NB), jnp.int32),
          pltpu.VMEM((_CAT, _NB, _D), jnp.float32),
          pltpu.VMEM((_NB, _D), jnp.float32),
          pltpu.SemaphoreType.DMA,
      ],
      compiler_params=pltpu.CompilerParams(
          use_tc_tiling_on_sc=False, needs_layout_passes=False),
  )(x, tables)


def _tc_body(xn_ref, w_ref, b_ref, cs_ref, o_ref):
  xn = xn_ref[...]
  w = w_ref[...]
  bias_sum = jnp.sum(b_ref[...], axis=0, keepdims=True)
  num = jnp.dot(xn, w, preferred_element_type=jnp.float32)
  o_ref[...] = (num + bias_sum + cs_ref[...]) * (1.0 / _F)


def _tc_finalize(x_num, num_weights, num_biases, cat_sum):
  b = x_num.shape[0]
  bt = 4096
  grid = b // bt
  return pl.pallas_call(
      _tc_body,
      grid=(grid,),
      in_specs=[
          pl.BlockSpec((bt, _NUM), lambda i: (i, 0)),
          pl.BlockSpec((_NUM, _D), lambda i: (0, 0)),
          pl.BlockSpec((_NUM, _D), lambda i: (0, 0)),
          pl.BlockSpec((bt, _D), lambda i: (i, 0)),
      ],
      out_specs=pl.BlockSpec((bt, _D), lambda i: (i, 0)),
      out_shape=jax.ShapeDtypeStruct((b, _D), jnp.float32),
  )(x_num, num_weights, num_biases, cat_sum)


@jax.jit
def kernel(x, tables, num_weights, num_biases):
  x_num = x[:, :_NUM]
  cat_sum = _sc_gather_sum(x, tables)
  return _tc_finalize(x_num, num_weights, num_biases, cat_sum)
